# Initial kernel scaffold; baseline (speedup 1.0000x reference)
import jax
import jax.numpy as jnp
from jax.experimental import pallas as pl


def kernel(inputs, params):
    raise NotImplementedError("write your pallas kernel here")


# speedup vs baseline: 3.7575x; 3.7575x over previous
import functools

import jax
import jax.numpy as jnp
from jax import lax
from jax.experimental import pallas as pl
from jax.experimental.pallas import tpu as pltpu
from jax.experimental.pallas import tpu_sc as plsc

B = 1024
HIST = 200
PRED = 50
WIN = HIST + PRED
TPAD = 256
D = 32
NB = 32
NW = 32

DENSE_IDS = (1, 2, 4, 6, 7, 8, 9)
HIST_DENSE = ((1, 0), (2, 2), (3, 4), (4, 5), (5, 6), (6, 7))


def _sc_embed(xp, idx3, idx5, idx0, e0, e3, e5, wd, bd):
    mesh = plsc.VectorSubcoreMesh(core_axis_name="c", subcore_axis_name="s")

    @functools.partial(
        pl.kernel,
        mesh=mesh,
        compiler_params=pltpu.CompilerParams(use_tc_tiling_on_sc=False),
        out_type=[
            jax.ShapeDtypeStruct((B, 2 * D), jnp.float32),
            jax.ShapeDtypeStruct((B, HIST, 8 * D), jnp.float32),
            jax.ShapeDtypeStruct((B, PRED, 2 * D), jnp.float32),
        ],
        scratch_types=[
            pltpu.VMEM((10, TPAD), jnp.int32),
            pltpu.VMEM((TPAD,), jnp.int32),
            pltpu.VMEM((TPAD,), jnp.int32),
            pltpu.VMEM((7, D), jnp.float32),
            pltpu.VMEM((7, D), jnp.float32),
            pltpu.VMEM((NB,), jnp.int32),
            pltpu.VMEM((TPAD, D), jnp.float32),
            pltpu.VMEM((208, D), jnp.float32),
            pltpu.VMEM((NB, D), jnp.float32),
            pltpu.VMEM((NB, 2 * D), jnp.float32),
            pltpu.VMEM((208, 8 * D), jnp.float32),
            pltpu.VMEM((64, 2 * D), jnp.float32),
            pltpu.SemaphoreType.DMA,
        ],
    )
    def k(xp_hbm, i3_hbm, i5_hbm, idx0_hbm, e0_hbm, e3_hbm, e5_hbm,
          wd_hbm, bd_hbm,
          st_out, hi_out, fu_out,
          inp_v, i3_v, i5_v, wd_v, bd_v, idx0_v, g3_v, g5_v, g0_v,
          st_v, hi_v, fu_v, sem):
        nc = 2
        wid = lax.axis_index("s") * nc + lax.axis_index("c")
        b0 = wid * NB

        pltpu.sync_copy(wd_hbm, wd_v)
        pltpu.sync_copy(bd_hbm, bd_v)
        wlo = [wd_v[k_, pl.ds(0, 16)] for k_ in range(7)]
        whi = [wd_v[k_, pl.ds(16, 16)] for k_ in range(7)]
        blo = [bd_v[k_, pl.ds(0, 16)] for k_ in range(7)]
        bhi = [bd_v[k_, pl.ds(16, 16)] for k_ in range(7)]

        def dense16(slot, xv):
            return (xv * wlo[slot] + blo[slot], xv * whi[slot] + bhi[slot])

        def bbody(i, carry):
            b = b0 + i
            pltpu.sync_copy(xp_hbm.at[b], inp_v)
            pltpu.sync_copy(i3_hbm.at[b], i3_v)
            pltpu.sync_copy(i5_hbm.at[b], i5_v)
            dmas = [
                pltpu.async_copy(e3_hbm.at[i3_v.at[pl.ds(0, 128)]],
                                 g3_v.at[pl.ds(0, 128)], sem),
                pltpu.async_copy(e3_hbm.at[i3_v.at[pl.ds(128, 128)]],
                                 g3_v.at[pl.ds(128, 128)], sem),
                pltpu.async_copy(e5_hbm.at[i5_v.at[pl.ds(0, 128)]],
                                 g5_v.at[pl.ds(0, 128)], sem),
                pltpu.async_copy(e5_hbm.at[i5_v.at[pl.ds(128, 80)]],
                                 g5_v.at[pl.ds(128, 80)], sem),
            ]

            def tbody(c, _):
                t0 = pl.multiple_of(c * 16, 16)
                for fid, col in HIST_DENSE:
                    xf = inp_v[DENSE_IDS[fid], pl.ds(t0, 16)].astype(jnp.float32)
                    for l in range(16):
                        xv = lax.broadcast_in_dim(xf[l], (16,), ())
                        lo, hi = dense16(fid, xv)
                        hi_v[t0 + l, pl.ds(col * D, 16)] = lo
                        hi_v[t0 + l, pl.ds(col * D + 16, 16)] = hi
                return _

            lax.fori_loop(0, 13, tbody, 0)

            def ubody(c, _):
                t0 = pl.multiple_of(192 + c * 16, 16)
                r0 = c * 16
                xf = inp_v[2, pl.ds(t0, 16)].astype(jnp.float32)
                for l in range(16):
                    xv = lax.broadcast_in_dim(xf[l], (16,), ())
                    lo, hi = dense16(1, xv)
                    fu_v[r0 + l, pl.ds(0, 16)] = lo
                    fu_v[r0 + l, pl.ds(16, 16)] = hi
                return _

            lax.fori_loop(0, 4, ubody, 0)

            x1 = inp_v[1, pl.ds(0, 16)].astype(jnp.float32)
            xv = lax.broadcast_in_dim(x1[0], (16,), ())
            lo, hi = dense16(0, xv)
            st_v[i, pl.ds(D, 16)] = lo
            st_v[i, pl.ds(D + 16, 16)] = hi

            for d_ in dmas:
                d_.wait()

            def cbody(c, _):
                t0 = pl.multiple_of(c * 16, 16)
                for l in range(16):
                    t = t0 + l
                    hi_v[t, pl.ds(1 * D, 16)] = g3_v[t, pl.ds(0, 16)]
                    hi_v[t, pl.ds(1 * D + 16, 16)] = g3_v[t, pl.ds(16, 16)]
                    hi_v[t, pl.ds(3 * D, 16)] = g5_v[t, pl.ds(0, 16)]
                    hi_v[t, pl.ds(3 * D + 16, 16)] = g5_v[t, pl.ds(16, 16)]
                return _

            lax.fori_loop(0, 13, cbody, 0)

            def vbody(c, _):
                r0 = c * 16
                for l in range(16):
                    r = r0 + l
                    fu_v[r, pl.ds(1 * D, 16)] = g3_v[192 + r, pl.ds(0, 16)]
                    fu_v[r, pl.ds(1 * D + 16, 16)] = g3_v[192 + r, pl.ds(16, 16)]
                return _

            lax.fori_loop(0, 4, vbody, 0)

            pltpu.sync_copy(hi_v.at[pl.ds(0, HIST)], hi_out.at[b])
            pltpu.sync_copy(fu_v.at[pl.ds(8, PRED)], fu_out.at[b])
            return carry

        lax.fori_loop(0, NB, bbody, 0)

        pltpu.sync_copy(idx0_hbm.at[pl.ds(b0, NB)], idx0_v)
        pltpu.async_copy(e0_hbm.at[idx0_v], g0_v, sem).wait()
        for c in range(2):
            for l in range(16):
                j = c * 16 + l
                st_v[j, pl.ds(0, 16)] = g0_v[j, pl.ds(0, 16)]
                st_v[j, pl.ds(16, 16)] = g0_v[j, pl.ds(16, 16)]
        pltpu.sync_copy(st_v, st_out.at[pl.ds(b0, NB)])

    return k(xp, idx3, idx5, idx0, e0, e3, e5, wd, bd)


def kernel(inputs, params):
    x = inputs
    if x.shape[1] > WIN:
        x = x[:, x.shape[1] - WIN:, :]
    xp = jnp.transpose(x, (0, 2, 1))
    xp = jnp.pad(xp, ((0, 0), (0, 0), (0, TPAD - WIN)))
    idx3 = xp[:, 3, :]
    idx5 = xp[:, 5, :]
    idx0 = x[:, 0, 0]
    wd = jnp.concatenate([params[f'W_{i}'] for i in DENSE_IDS], axis=0)
    bd = jnp.stack([params[f'b_{i}'] for i in DENSE_IDS], axis=0)
    st, hi, fu = _sc_embed(xp, idx3, idx5, idx0, params['emb_0'],
                           params['emb_3'], params['emb_5'], wd, bd)
    return (st.reshape(B, 1, 2, D),
            hi.reshape(B, HIST, 8, D),
            fu.reshape(B, PRED, 2, D))


# speedup vs baseline: 3.9847x; 1.0605x over previous
import functools

import jax
import jax.numpy as jnp
from jax import lax
from jax.experimental import pallas as pl
from jax.experimental.pallas import tpu as pltpu
from jax.experimental.pallas import tpu_sc as plsc

B = 1024
HIST = 200
PRED = 50
WIN = HIST + PRED
TPAD = 256
D = 32
NB = 32
H0 = 112
H1 = 96

DENSE_IDS = (1, 2, 4, 6, 7, 8, 9)
HIST_DENSE = ((1, 0), (2, 2), (3, 4), (4, 5), (5, 6), (6, 7))


def _sc_embed(xp, idx3, idx5, idx0, e0, e3, e5, wd, bd):
    mesh = plsc.VectorSubcoreMesh(core_axis_name="c", subcore_axis_name="s")

    @functools.partial(
        pl.kernel,
        mesh=mesh,
        compiler_params=pltpu.CompilerParams(use_tc_tiling_on_sc=False),
        out_type=[
            jax.ShapeDtypeStruct((B, 2, D), jnp.float32),
            jax.ShapeDtypeStruct((B, HIST, 8 * D), jnp.float32),
            jax.ShapeDtypeStruct((B, PRED, 2 * D), jnp.float32),
        ],
        scratch_types=[
            pltpu.VMEM((10, TPAD), jnp.int32),
            pltpu.VMEM((TPAD,), jnp.int32),
            pltpu.VMEM((TPAD,), jnp.int32),
            pltpu.VMEM((TPAD, D), jnp.float32),
            pltpu.VMEM((208, D), jnp.float32),
            pltpu.VMEM((H0, 8 * D), jnp.float32),
            pltpu.VMEM((H1, 8 * D), jnp.float32),
            pltpu.VMEM((64, 2 * D), jnp.float32),
            pltpu.VMEM((64, 2 * D), jnp.float32),
            pltpu.VMEM((NB,), jnp.int32),
            pltpu.VMEM((NB, D), jnp.float32),
            pltpu.VMEM((NB, D), jnp.float32),
            pltpu.VMEM((7, D), jnp.float32),
            pltpu.VMEM((7, D), jnp.float32),
            pltpu.SemaphoreType.DMA,
            pltpu.SemaphoreType.DMA,
            pltpu.SemaphoreType.DMA,
            pltpu.SemaphoreType.DMA,
            pltpu.SemaphoreType.DMA,
            pltpu.SemaphoreType.DMA,
        ],
    )
    def k(xp_hbm, i3_hbm, i5_hbm, idx0_hbm, e0_hbm, e3_hbm, e5_hbm,
          wd_hbm, bd_hbm,
          st_out, hi_out, fu_out,
          inp_v, i3_v, i5_v, g3_v, g5_v, asm0, asm1, fua, fub,
          idx0_v, st1_v, g0_v, wd_v, bd_v,
          sem_g1, sem_g2, sem_h0, sem_h1, sem_fa, sem_fb):
        nc = 2
        wid = lax.axis_index("s") * nc + lax.axis_index("c")
        b0 = wid * NB

        pltpu.sync_copy(wd_hbm, wd_v)
        pltpu.sync_copy(bd_hbm, bd_v)
        wlo = [wd_v[k_, pl.ds(0, 16)] for k_ in range(7)]
        whi = [wd_v[k_, pl.ds(16, 16)] for k_ in range(7)]
        blo = [bd_v[k_, pl.ds(0, 16)] for k_ in range(7)]
        bhi = [bd_v[k_, pl.ds(16, 16)] for k_ in range(7)]

        def h0_out(b):
            return pltpu.make_async_copy(
                asm0, hi_out.at[b, pl.ds(0, H0)], sem_h0)

        def h1_out(b):
            return pltpu.make_async_copy(
                asm1.at[pl.ds(0, HIST - H0)], hi_out.at[b, pl.ds(H0, HIST - H0)],
                sem_h1)

        def fu_out_d(b, fu_v, sem_f):
            return pltpu.make_async_copy(
                fu_v.at[pl.ds(8, PRED)], fu_out.at[b], sem_f)

        def body(i, b, fu_v, sem_f, drain_fu, drain_h):
            pltpu.sync_copy(xp_hbm.at[b], inp_v)
            pltpu.sync_copy(i3_hbm.at[b], i3_v)
            pltpu.sync_copy(i5_hbm.at[b], i5_v)

            g1 = [
                pltpu.async_copy(e3_hbm.at[i3_v.at[pl.ds(0, 128)]],
                                 g3_v.at[pl.ds(0, 128)], sem_g1),
                pltpu.async_copy(e5_hbm.at[i5_v.at[pl.ds(0, 128)]],
                                 g5_v.at[pl.ds(0, 128)], sem_g1),
            ]
            g2 = [
                pltpu.async_copy(e3_hbm.at[i3_v.at[pl.ds(128, 128)]],
                                 g3_v.at[pl.ds(128, 128)], sem_g2),
                pltpu.async_copy(e5_hbm.at[i5_v.at[pl.ds(128, 80)]],
                                 g5_v.at[pl.ds(128, 80)], sem_g2),
            ]

            @pl.when(drain_h)
            def _():
                h0_out(b).wait()

            def t0body(c, _):
                t0 = pl.multiple_of(c * 16, 16)
                for s, (fid, col) in enumerate(HIST_DENSE):
                    xf = inp_v[DENSE_IDS[fid], pl.ds(t0, 16)].astype(jnp.float32)
                    for l in range(16):
                        xv = lax.broadcast_in_dim(xf[l], (16,), ())
                        asm0[t0 + l, pl.ds(col * D, 16)] = xv * wlo[fid] + blo[fid]
                        asm0[t0 + l, pl.ds(col * D + 16, 16)] = xv * whi[fid] + bhi[fid]
                return _

            lax.fori_loop(0, 7, t0body, 0)

            for d_ in g1:
                d_.wait()

            def c0body(c, _):
                t0 = pl.multiple_of(c * 16, 16)
                for l in range(16):
                    t = t0 + l
                    asm0[t, pl.ds(1 * D, 16)] = g3_v[t, pl.ds(0, 16)]
                    asm0[t, pl.ds(1 * D + 16, 16)] = g3_v[t, pl.ds(16, 16)]
                    asm0[t, pl.ds(3 * D, 16)] = g5_v[t, pl.ds(0, 16)]
                    asm0[t, pl.ds(3 * D + 16, 16)] = g5_v[t, pl.ds(16, 16)]
                return _

            lax.fori_loop(0, 7, c0body, 0)
            h0_out(b).start()

            @pl.when(drain_h)
            def _():
                h1_out(b).wait()

            def t1body(c, _):
                r0 = pl.multiple_of(c * 16, 16)
                t0 = pl.multiple_of(H0 + c * 16, 16)
                for s, (fid, col) in enumerate(HIST_DENSE):
                    xf = inp_v[DENSE_IDS[fid], pl.ds(t0, 16)].astype(jnp.float32)
                    for l in range(16):
                        xv = lax.broadcast_in_dim(xf[l], (16,), ())
                        asm1[r0 + l, pl.ds(col * D, 16)] = xv * wlo[fid] + blo[fid]
                        asm1[r0 + l, pl.ds(col * D + 16, 16)] = xv * whi[fid] + bhi[fid]
                return _

            lax.fori_loop(0, 6, t1body, 0)

            for d_ in g2:
                d_.wait()

            def c1body(c, _):
                r0 = pl.multiple_of(c * 16, 16)
                for l in range(16):
                    r = r0 + l
                    asm1[r, pl.ds(1 * D, 16)] = g3_v[H0 + r, pl.ds(0, 16)]
                    asm1[r, pl.ds(1 * D + 16, 16)] = g3_v[H0 + r, pl.ds(16, 16)]
                    asm1[r, pl.ds(3 * D, 16)] = g5_v[H0 + r, pl.ds(0, 16)]
                    asm1[r, pl.ds(3 * D + 16, 16)] = g5_v[H0 + r, pl.ds(16, 16)]
                return _

            lax.fori_loop(0, 6, c1body, 0)
            h1_out(b).start()

            @pl.when(drain_fu)
            def _():
                fu_out_d(b, fu_v, sem_f).wait()

            def ubody(c, _):
                t0 = pl.multiple_of(192 + c * 16, 16)
                r0 = c * 16
                xf = inp_v[2, pl.ds(t0, 16)].astype(jnp.float32)
                for l in range(16):
                    xv = lax.broadcast_in_dim(xf[l], (16,), ())
                    fu_v[r0 + l, pl.ds(0, 16)] = xv * wlo[1] + blo[1]
                    fu_v[r0 + l, pl.ds(16, 16)] = xv * whi[1] + bhi[1]
                    fu_v[r0 + l, pl.ds(1 * D + 0, 16)] = g3_v[192 + r0 + l, pl.ds(0, 16)]
                    fu_v[r0 + l, pl.ds(1 * D + 16, 16)] = g3_v[192 + r0 + l, pl.ds(16, 16)]
                return _

            lax.fori_loop(0, 4, ubody, 0)
            fu_out_d(b, fu_v, sem_f).start()

            x1 = inp_v[1, pl.ds(0, 16)].astype(jnp.float32)
            xv = lax.broadcast_in_dim(x1[0], (16,), ())
            st1_v[i, pl.ds(0, 16)] = xv * wlo[0] + blo[0]
            st1_v[i, pl.ds(16, 16)] = xv * whi[0] + bhi[0]

        def bbody(j, carry):
            ba = b0 + 2 * j
            body(2 * j, ba, fua, sem_fa, j > 0, j > 0)
            body(2 * j + 1, ba + 1, fub, sem_fb, j > 0, True)
            return carry

        lax.fori_loop(0, NB // 2, bbody, 0)

        h0_out(b0).wait()
        h1_out(b0).wait()
        fu_out_d(b0, fua, sem_fa).wait()
        fu_out_d(b0, fub, sem_fb).wait()

        pltpu.sync_copy(idx0_hbm.at[pl.ds(b0, NB)], idx0_v)
        pltpu.async_copy(e0_hbm.at[idx0_v], g0_v, sem_g1).wait()
        pltpu.sync_copy(g0_v, st_out.at[pl.ds(b0, NB), 0, :])
        pltpu.sync_copy(st1_v, st_out.at[pl.ds(b0, NB), 1, :])

    return k(xp, idx3, idx5, idx0, e0, e3, e5, wd, bd)


def kernel(inputs, params):
    x = inputs
    if x.shape[1] > WIN:
        x = x[:, x.shape[1] - WIN:, :]
    xp = jnp.transpose(x, (0, 2, 1))
    xp = jnp.pad(xp, ((0, 0), (0, 0), (0, TPAD - WIN)))
    idx3 = xp[:, 3, :]
    idx5 = xp[:, 5, :]
    idx0 = x[:, 0, 0]
    wd = jnp.concatenate([params[f'W_{i}'] for i in DENSE_IDS], axis=0)
    bd = jnp.stack([params[f'b_{i}'] for i in DENSE_IDS], axis=0)
    st, hi, fu = _sc_embed(xp, idx3, idx5, idx0, params['emb_0'],
                           params['emb_3'], params['emb_5'], wd, bd)
    return (st.reshape(B, 1, 2, D),
            hi.reshape(B, HIST, 8, D),
            fu.reshape(B, PRED, 2, D))
